# baseline (speedup 1.0000x reference)
import jax
import jax.numpy as jnp
from jax import lax
from jax.experimental import pallas as pl
from jax.experimental.pallas import tpu as pltpu
from jax.experimental.pallas import tpu_sc as plsc

N_NODES = 100000
EMB_DIM = 50
N_EDGES = 1600000
EPS = 1e-7
MIN_NORM = 1e-15

NSC = 2
NSUB = 16
NCHUNK = 4
NLAYER = 3
TILE = 768
MB = 192
NMB = TILE // MB
TILES_PER_SUB = 134
EDGES_PER_SUB = TILES_PER_SUB * TILE
E_PAD = EDGES_PER_SUB * NSUB
N_PAD = 100096
ACC_ROWS = N_PAD // NSUB
TROWS = 3128


def _spmm3_body(xf, pack, zhbm, o1, o2, o3,
                pbuf0, pbuf1, rows0, rows1,
                acc, isem0, isem1, gsem0, gsem1, ssem0, ssem1):
    cidx = lax.axis_index("c")
    sid = lax.axis_index("s")
    base = sid * ACC_ROWS
    t0 = sid * TILES_PER_SUB
    pbufs = (pbuf0, pbuf1)
    rowss = (rows0, rows1)
    isems = (isem0, isem1)
    gsems = (gsem0, gsem1)
    ssems = (ssem0, ssem1)
    nt = TILES_PER_SUB

    def in_desc(c, t, b):
        return pltpu.make_async_copy(pack.at[c * (NSUB * TILES_PER_SUB) + t0 + t],
                                     pbufs[b], isems[b])

    def scat_descs(b):
        return [
            pltpu.make_async_copy(
                rowss[b].at[pl.ds(j * MB, MB)],
                acc.at[pbufs[b].at[pl.ds(2 * TILE + j * MB, MB)]], ssems[b])
            for j in range(NMB)
        ]

    for layer in range(NLAYER):
        srcx = (xf, o1, o2)[layer]
        outx = (o1, o2, o3)[layer]
        for step in range(NCHUNK // NSC):
            c = step * NSC + cidx
            cn = c * N_PAD
            pltpu.sync_copy(zhbm, acc.at[pl.ds(base, ACC_ROWS)])
            plsc.subcore_barrier()

            in_desc(c, 0, 0).start()

            def pair_body(k, carry):
                for b in (0, 1):
                    t = 2 * k + b
                    in_desc(c, t, b).wait()

                    gathers = [
                        pltpu.async_copy(
                            srcx.at[pbufs[b].at[pl.ds(j * MB, MB)]],
                            rowss[b].at[pl.ds(j * MB, MB)], gsems[b])
                        for j in range(NMB)
                    ]

                    @pl.when(t >= 1)
                    def _():
                        for d in scat_descs(1 - b):
                            d.wait()

                    @pl.when(t < nt - 1)
                    def _():
                        in_desc(c, t + 1, 1 - b).start()

                    for cp in gathers:
                        cp.wait()

                    rw = rowss[b]
                    pb = pbufs[b]

                    def mul(g, carry2):
                        wv16 = plsc.bitcast(
                            pb[pl.ds(TILE + g * 16, 16)], jnp.float32)
                        for l in range(16):
                            e = g * 16 + l
                            rw[e, :] = rw[e, :] * wv16[l]
                        return carry2
                    lax.fori_loop(0, TILE // 16, mul, 0)

                    for d in scat_descs(b):
                        d.start(add=True)
                return carry
            lax.fori_loop(0, nt // 2, pair_body, 0)
            for d in scat_descs((nt - 1) % 2):
                d.wait()
            plsc.subcore_barrier()
            pltpu.sync_copy(acc.at[pl.ds(base, ACC_ROWS)],
                            outx.at[pl.ds(cn + base, ACC_ROWS)])
            plsc.subcore_barrier()


_ods = jax.ShapeDtypeStruct((NCHUNK * N_PAD, 16), jnp.float32)
_spmm3_sc = pl.kernel(
    _spmm3_body,
    out_type=(_ods, _ods, _ods),
    mesh=plsc.VectorSubcoreMesh(core_axis_name="c", subcore_axis_name="s"),
    compiler_params=pltpu.CompilerParams(use_tc_tiling_on_sc=False,
                                         needs_layout_passes=False),
    scratch_types=[
        pltpu.VMEM((3 * TILE,), jnp.int32),
        pltpu.VMEM((3 * TILE,), jnp.int32),
        pltpu.VMEM((TILE, 16), jnp.float32),
        pltpu.VMEM((TILE, 16), jnp.float32),
        pltpu.VMEM_SHARED((N_PAD, 16), jnp.float32),
        pltpu.SemaphoreType.DMA,
        pltpu.SemaphoreType.DMA,
        pltpu.SemaphoreType.DMA,
        pltpu.SemaphoreType.DMA,
        pltpu.SemaphoreType.DMA,
        pltpu.SemaphoreType.DMA,
    ],
)


def _tangent_body(w_ref, o_ref):
    w = w_ref[...]
    y = w[:, 1:]
    y_sq = jnp.sum(y * y, axis=1, keepdims=True)
    x0 = jnp.sqrt(jnp.clip(1.0 + y_sq, EPS, None))
    y_norm = jnp.clip(jnp.sqrt(y_sq), MIN_NORM, None)
    theta = jnp.clip(x0, 1.0 + EPS, None)
    acosh = jnp.log(theta + jnp.sqrt(theta * theta - 1.0))
    rest = acosh * y / y_norm
    x64 = jnp.concatenate(
        [jnp.zeros_like(x0), rest,
         jnp.zeros((rest.shape[0], NCHUNK * 16 - EMB_DIM), rest.dtype)],
        axis=1)
    for cc in range(NCHUNK):
        o_ref[cc, :, :] = x64[:, cc * 16:(cc + 1) * 16]


def _decode_body(h1_ref, h2_ref, h3_ref, o_ref):
    h64 = jnp.concatenate(
        [h1_ref[cc, :, :] + h2_ref[cc, :, :] + h3_ref[cc, :, :]
         for cc in range(NCHUNK)], axis=1)
    x = h64[:, 1:EMB_DIM]
    x_sq = jnp.sum(x * x, axis=1, keepdims=True)
    x_norm = jnp.clip(jnp.sqrt(x_sq), MIN_NORM, None)
    e = jnp.exp(x_norm)
    ei = 1.0 / e
    sinh = 0.5 * (e - ei)
    rest = sinh * x / x_norm
    r_sq = jnp.sum(rest * rest, axis=1, keepdims=True)
    x0 = jnp.sqrt(jnp.clip(1.0 + r_sq, EPS, None))
    o_ref[...] = jnp.concatenate([x0, rest], axis=1)


_tangent_tc = pl.pallas_call(
    _tangent_body,
    grid=(N_PAD // TROWS,),
    in_specs=[pl.BlockSpec((TROWS, EMB_DIM), lambda i: (i, 0))],
    out_specs=pl.BlockSpec((NCHUNK, TROWS, 16), lambda i: (0, i, 0)),
    out_shape=jax.ShapeDtypeStruct((NCHUNK, N_PAD, 16), jnp.float32),
)

_decode_tc = pl.pallas_call(
    _decode_body,
    grid=(N_PAD // TROWS,),
    in_specs=[pl.BlockSpec((NCHUNK, TROWS, 16), lambda i: (0, i, 0))
              for _ in range(NLAYER)],
    out_specs=pl.BlockSpec((TROWS, EMB_DIM), lambda i: (i, 0)),
    out_shape=jax.ShapeDtypeStruct((N_NODES, EMB_DIM), jnp.float32),
)


@jax.jit
def kernel(weight, edge_index, edge_weight):
    pad = E_PAD - N_EDGES
    src = jnp.pad(edge_index[0].astype(jnp.int32), (0, pad))
    srcs4 = src[None, :] + (jnp.arange(NCHUNK, dtype=jnp.int32)
                            * N_PAD)[:, None]
    dst = jnp.pad(edge_index[1].astype(jnp.int32), (0, pad))
    wbits = jax.lax.bitcast_convert_type(jnp.pad(edge_weight, (0, pad)),
                                         jnp.int32)
    pack = jnp.stack(
        [srcs4.reshape(NCHUNK, E_PAD // TILE, TILE),
         jnp.broadcast_to(wbits.reshape(1, E_PAD // TILE, TILE),
                          (NCHUNK, E_PAD // TILE, TILE)),
         jnp.broadcast_to(dst.reshape(1, E_PAD // TILE, TILE),
                          (NCHUNK, E_PAD // TILE, TILE))], axis=2).reshape(
        NCHUNK * (E_PAD // TILE), 3 * TILE)
    zhbm = jnp.zeros((ACC_ROWS, 16), jnp.float32)

    xf = _tangent_tc(weight).reshape(NCHUNK * N_PAD, 16)
    o1, o2, o3 = _spmm3_sc(xf, pack, zhbm)
    return _decode_tc(o1.reshape(NCHUNK, N_PAD, 16),
                      o2.reshape(NCHUNK, N_PAD, 16),
                      o3.reshape(NCHUNK, N_PAD, 16))


# speedup vs baseline: 1.0769x; 1.0769x over previous
import jax
import jax.numpy as jnp
from jax import lax
from jax.experimental import pallas as pl
from jax.experimental.pallas import tpu as pltpu
from jax.experimental.pallas import tpu_sc as plsc

N_NODES = 100000
EMB_DIM = 50
N_EDGES = 1600000
EPS = 1e-7
MIN_NORM = 1e-15

NSC = 2
NSUB = 16
NCHUNK = 4
NLAYER = 3
TILE = 512
MB = 256
NMB = TILE // MB
TILES_PER_SUB = 200
EDGES_PER_SUB = TILES_PER_SUB * TILE
E_PAD = EDGES_PER_SUB * NSUB
N_PAD = 100096
ACC_ROWS = N_PAD // NSUB
TROWS = 3128


def _spmm3_body(xf, pack, wv, zhbm, o1, o2, o3,
                pbuf0, pbuf1, wbuf0, wbuf1, rows0, rows1,
                acc, isem0, isem1, gsem0, gsem1, ssem0, ssem1):
    cidx = lax.axis_index("c")
    sid = lax.axis_index("s")
    base = sid * ACC_ROWS
    t0 = sid * TILES_PER_SUB
    pbufs = (pbuf0, pbuf1)
    wbufs = (wbuf0, wbuf1)
    rowss = (rows0, rows1)
    isems = (isem0, isem1)
    gsems = (gsem0, gsem1)
    ssems = (ssem0, ssem1)
    nt = TILES_PER_SUB

    def in_descs(c, t, b):
        return (
            pltpu.make_async_copy(
                pack.at[c * (NSUB * TILES_PER_SUB) + t0 + t], pbufs[b],
                isems[b]),
            pltpu.make_async_copy(
                wv.at[pl.ds((t0 + t) * TILE, TILE)], wbufs[b], isems[b]),
        )

    def scat_descs(b):
        return [
            pltpu.make_async_copy(
                rowss[b].at[pl.ds(j * MB, MB)],
                acc.at[pbufs[b].at[pl.ds(TILE + j * MB, MB)]], ssems[b])
            for j in range(NMB)
        ]

    for layer in range(NLAYER):
        srcx = (xf, o1, o2)[layer]
        outx = (o1, o2, o3)[layer]
        for step in range(NCHUNK // NSC):
            c = step * NSC + cidx
            cn = c * N_PAD
            pltpu.sync_copy(zhbm, acc.at[pl.ds(base, ACC_ROWS)])
            plsc.subcore_barrier()

            for d in in_descs(c, 0, 0):
                d.start()

            def pair_body(k, carry):
                for b in (0, 1):
                    t = 2 * k + b
                    for d in in_descs(c, t, b):
                        d.wait()

                    gathers = [
                        pltpu.async_copy(
                            srcx.at[pbufs[b].at[pl.ds(j * MB, MB)]],
                            rowss[b].at[pl.ds(j * MB, MB)], gsems[b])
                        for j in range(NMB)
                    ]

                    @pl.when(t >= 1)
                    def _():
                        for d in scat_descs(1 - b):
                            d.wait()

                    @pl.when(t < nt - 1)
                    def _():
                        for d in in_descs(c, t + 1, 1 - b):
                            d.start()

                    for cp in gathers:
                        cp.wait()

                    rw = rowss[b]
                    wb = wbufs[b]

                    def mul(g, carry2):
                        wv16 = wb[pl.ds(g * 16, 16)]
                        for l in range(16):
                            e = g * 16 + l
                            rw[e, :] = rw[e, :] * wv16[l]
                        return carry2
                    lax.fori_loop(0, TILE // 16, mul, 0)

                    for d in scat_descs(b):
                        d.start(add=True)
                return carry
            lax.fori_loop(0, nt // 2, pair_body, 0)
            for d in scat_descs((nt - 1) % 2):
                d.wait()
            plsc.subcore_barrier()
            pltpu.sync_copy(acc.at[pl.ds(base, ACC_ROWS)],
                            outx.at[pl.ds(cn + base, ACC_ROWS)])
            plsc.subcore_barrier()


_ods = jax.ShapeDtypeStruct((NCHUNK * N_PAD, 16), jnp.float32)
_spmm3_sc = pl.kernel(
    _spmm3_body,
    out_type=(_ods, _ods, _ods),
    mesh=plsc.VectorSubcoreMesh(core_axis_name="c", subcore_axis_name="s"),
    compiler_params=pltpu.CompilerParams(use_tc_tiling_on_sc=False),
    scratch_types=[
        pltpu.VMEM((2 * TILE,), jnp.int32),
        pltpu.VMEM((2 * TILE,), jnp.int32),
        pltpu.VMEM((TILE,), jnp.float32),
        pltpu.VMEM((TILE,), jnp.float32),
        pltpu.VMEM((TILE, 16), jnp.float32),
        pltpu.VMEM((TILE, 16), jnp.float32),
        pltpu.VMEM_SHARED((N_PAD, 16), jnp.float32),
        pltpu.SemaphoreType.DMA,
        pltpu.SemaphoreType.DMA,
        pltpu.SemaphoreType.DMA,
        pltpu.SemaphoreType.DMA,
        pltpu.SemaphoreType.DMA,
        pltpu.SemaphoreType.DMA,
    ],
)


def _tangent_body(w_ref, o_ref):
    w = w_ref[...]
    y = w[:, 1:]
    y_sq = jnp.sum(y * y, axis=1, keepdims=True)
    x0 = jnp.sqrt(jnp.clip(1.0 + y_sq, EPS, None))
    y_norm = jnp.clip(jnp.sqrt(y_sq), MIN_NORM, None)
    theta = jnp.clip(x0, 1.0 + EPS, None)
    acosh = jnp.log(theta + jnp.sqrt(theta * theta - 1.0))
    rest = acosh * y / y_norm
    x64 = jnp.concatenate(
        [jnp.zeros_like(x0), rest,
         jnp.zeros((rest.shape[0], NCHUNK * 16 - EMB_DIM), rest.dtype)],
        axis=1)
    for cc in range(NCHUNK):
        o_ref[cc, :, :] = x64[:, cc * 16:(cc + 1) * 16]


def _decode_body(h1_ref, h2_ref, h3_ref, o_ref):
    h64 = jnp.concatenate(
        [h1_ref[cc, :, :] + h2_ref[cc, :, :] + h3_ref[cc, :, :]
         for cc in range(NCHUNK)], axis=1)
    x = h64[:, 1:EMB_DIM]
    x_sq = jnp.sum(x * x, axis=1, keepdims=True)
    x_norm = jnp.clip(jnp.sqrt(x_sq), MIN_NORM, None)
    e = jnp.exp(x_norm)
    ei = 1.0 / e
    sinh = 0.5 * (e - ei)
    rest = sinh * x / x_norm
    r_sq = jnp.sum(rest * rest, axis=1, keepdims=True)
    x0 = jnp.sqrt(jnp.clip(1.0 + r_sq, EPS, None))
    o_ref[...] = jnp.concatenate([x0, rest], axis=1)


_tangent_tc = pl.pallas_call(
    _tangent_body,
    grid=(N_PAD // TROWS,),
    in_specs=[pl.BlockSpec((TROWS, EMB_DIM), lambda i: (i, 0))],
    out_specs=pl.BlockSpec((NCHUNK, TROWS, 16), lambda i: (0, i, 0)),
    out_shape=jax.ShapeDtypeStruct((NCHUNK, N_PAD, 16), jnp.float32),
)

_decode_tc = pl.pallas_call(
    _decode_body,
    grid=(N_PAD // TROWS,),
    in_specs=[pl.BlockSpec((NCHUNK, TROWS, 16), lambda i: (0, i, 0))
              for _ in range(NLAYER)],
    out_specs=pl.BlockSpec((TROWS, EMB_DIM), lambda i: (i, 0)),
    out_shape=jax.ShapeDtypeStruct((N_NODES, EMB_DIM), jnp.float32),
)


@jax.jit
def kernel(weight, edge_index, edge_weight):
    pad = E_PAD - N_EDGES
    src = jnp.pad(edge_index[0].astype(jnp.int32), (0, pad))
    srcs4 = src[None, :] + (jnp.arange(NCHUNK, dtype=jnp.int32)
                            * N_PAD)[:, None]
    dst = jnp.pad(edge_index[1].astype(jnp.int32), (0, pad))
    pack = jnp.stack(
        [srcs4.reshape(NCHUNK, E_PAD // TILE, TILE),
         jnp.broadcast_to(dst.reshape(1, E_PAD // TILE, TILE),
                          (NCHUNK, E_PAD // TILE, TILE))], axis=2).reshape(
        NCHUNK * (E_PAD // TILE), 2 * TILE)
    w = jnp.pad(edge_weight, (0, pad))
    zhbm = jnp.zeros((ACC_ROWS, 16), jnp.float32)

    xf = _tangent_tc(weight).reshape(NCHUNK * N_PAD, 16)
    o1, o2, o3 = _spmm3_sc(xf, pack, w, zhbm)
    return _decode_tc(o1.reshape(NCHUNK, N_PAD, 16),
                      o2.reshape(NCHUNK, N_PAD, 16),
                      o3.reshape(NCHUNK, N_PAD, 16))
